# baseline (speedup 1.0000x reference)
import functools

import jax
import jax.numpy as jnp
from jax import lax
from jax.experimental import pallas as pl
from jax.experimental.pallas import tpu as pltpu
from jax.experimental.pallas import tpu_sc as plsc

_B, _C, _H, _W = 16, 64, 32, 32
_HW = _H * _W
_K = 1024
_NTOK = _B * _HW
_LOSS_SCALE = 1.25 / (_B * _C * _HW)
_BPS = 4
_CP = 128


def _vq_idx_body(z_ref, emb_ref, idx_ref, loss_ref):
    emb = emb_ref[...]
    esq = jnp.sum(emb * emb, axis=1)
    esqc = esq[:, None]
    emb2 = emb + emb

    part = jnp.float32(0.0)
    for j in range(_BPS):
        zb = z_ref[j]

        mm2 = jax.lax.dot_general(
            emb2, zb, (((1,), (0,)), ((), ())))
        zsq_t = (zb * zb).T
        s1row = jnp.sum(zsq_t, axis=1, keepdims=True)
        s1 = s1row.T
        d = (s1 - mm2) + esqc

        dmin = jnp.min(d, axis=0, keepdims=True)
        iota_k = jax.lax.broadcasted_iota(jnp.int32, (_K, _HW), 0)
        idx = jnp.min(jnp.where(d == dmin, iota_k, _K),
                      axis=0, keepdims=True)
        idx_ref[j] = idx

        part = part + jnp.sum(dmin)

    loss_ref[0, 0, 0] = part * _LOSS_SCALE


def _vq_out_body(q_ref, z_ref, out_ref):
    for j in range(_BPS):
        zb = z_ref[j]
        qT = q_ref[j][:, :_C].T
        out_ref[j] = zb + (qT - zb)


def _make_sc_gather():
    info = plsc.get_sparse_core_info()
    nc, ns = info.num_cores, info.num_subcores
    nw = nc * ns
    b_per_w = _NTOK // nw
    mesh = plsc.VectorSubcoreMesh(core_axis_name="c", subcore_axis_name="s")

    @functools.partial(
        pl.kernel, mesh=mesh,
        out_type=jax.ShapeDtypeStruct((_NTOK, _CP), jnp.float32),
        scratch_types=[
            pltpu.VMEM((b_per_w,), jnp.int32),
            pltpu.VMEM((b_per_w, _CP), jnp.float32),
            pltpu.SemaphoreType.DMA,
        ],
    )
    def _sc_gather(table_hbm, idx_hbm, out_hbm, idx_v, rows_v, sem):
        wid = lax.axis_index("s") * nc + lax.axis_index("c")
        base = wid * b_per_w
        pltpu.sync_copy(idx_hbm.at[pl.ds(base, b_per_w)], idx_v)
        pltpu.async_copy(table_hbm.at[idx_v], rows_v, sem).wait()
        pltpu.sync_copy(rows_v, out_hbm.at[pl.ds(base, b_per_w)])

    return _sc_gather


@jax.jit
def _vq(z3, embeddings):
    idx3, loss = pl.pallas_call(
        _vq_idx_body,
        grid=(_B // _BPS,),
        in_specs=[
            pl.BlockSpec((_BPS, _C, _HW), lambda b: (b, 0, 0)),
            pl.BlockSpec((_K, _C), lambda b: (0, 0)),
        ],
        out_specs=[
            pl.BlockSpec((_BPS, 1, _HW), lambda b: (b, 0, 0)),
            pl.BlockSpec((1, 1, 1), lambda b: (b, 0, 0), memory_space=pltpu.SMEM),
        ],
        out_shape=[
            jax.ShapeDtypeStruct((_B, 1, _HW), jnp.int32),
            jax.ShapeDtypeStruct((_B // _BPS, 1, 1), jnp.float32),
        ],
    )(z3, embeddings)

    embp = jnp.pad(embeddings, ((0, 0), (0, _CP - _C)))
    q_tok = _make_sc_gather()(embp, idx3.reshape(_NTOK))

    out = pl.pallas_call(
        _vq_out_body,
        grid=(_B // _BPS,),
        in_specs=[
            pl.BlockSpec((_BPS, _HW, _CP), lambda b: (b, 0, 0)),
            pl.BlockSpec((_BPS, _C, _HW), lambda b: (b, 0, 0)),
        ],
        out_specs=pl.BlockSpec((_BPS, _C, _HW), lambda b: (b, 0, 0)),
        out_shape=jax.ShapeDtypeStruct((_B, _C, _HW), jnp.float32),
    )(q_tok.reshape(_B, _HW, _CP), z3)

    return out, jnp.sum(loss)


def kernel(z, embeddings):
    z3 = z.reshape(_B, _C, _HW)
    out, loss = _vq(z3, embeddings)
    return out.reshape(_B, _C, _H, _W), loss
